# baseline (speedup 1.0000x reference)
import functools

import jax
import jax.numpy as jnp
from jax import lax
from jax.experimental import pallas as pl
from jax.experimental.pallas import tpu as pltpu
from jax.experimental.pallas import tpu_sc as plsc

_BATCH = 4096
_MAX_LEN = 200
_DIM = 50
_DPAD = 64
_HALF = _BATCH * _MAX_LEN
_NW = 32
_IW = 128
_IROWS_W = _HALF // _NW // _IW
_BLK = 4
_CHUNK = _BLK * _IW
_NCHUNK = _IROWS_W // _BLK
_CWORDS = _CHUNK * _DIM


def _make_gather():
    mesh = plsc.VectorSubcoreMesh(core_axis_name="c", subcore_axis_name="s")

    @functools.partial(
        pl.kernel,
        out_type=jax.ShapeDtypeStruct((2 * _HALF * _DIM,), jnp.float32),
        mesh=mesh,
        scratch_types=[
            pltpu.VMEM((_BLK, _IW), jnp.int32),
            pltpu.VMEM((_CHUNK, _DPAD), jnp.float32),
            pltpu.VMEM((_CWORDS + 16,), jnp.float32),
            pltpu.SemaphoreType.DMA,
        ],
        compiler_params=pltpu.CompilerParams(use_tc_tiling_on_sc=False),
    )
    def gather(idx_t_hbm, idx_j_hbm, table_hbm, out_hbm, idx_v, win_v,
               rows_v, sem):
        wid = lax.axis_index("s") * 2 + lax.axis_index("c")
        irow_base = wid * _IROWS_W
        for h, idx_hbm in enumerate((idx_t_hbm, idx_j_hbm)):
            h_out = (h * _HALF + wid * _IROWS_W * _IW) * _DIM

            @pl.loop(0, _NCHUNK)
            def _chunk(c, idx_hbm=idx_hbm, h_out=h_out):
                pltpu.sync_copy(
                    idx_hbm.at[pl.ds(irow_base + c * _BLK, _BLK)], idx_v
                )
                copies = [
                    pltpu.async_copy(
                        table_hbm.at[idx_v.at[j]],
                        win_v.at[pl.ds(j * _IW, _IW)],
                        sem,
                    )
                    for j in range(_BLK)
                ]
                for cp in copies:
                    cp.wait()

                @pl.loop(0, _CHUNK, unroll=4)
                def _compact(r):
                    for m in range(4):
                        rows_v[pl.ds(_DIM * r + 16 * m, 16)] = (
                            win_v[r, pl.ds(16 * m, 16)]
                        )

                pltpu.sync_copy(
                    rows_v.at[pl.ds(0, _CWORDS)],
                    out_hbm.at[pl.ds(h_out + c * _CWORDS, _CWORDS)],
                )

    return gather


_gather = _make_gather()


def kernel(idx_t, idx_j, emb_matrix):
    table_padded = jnp.pad(emb_matrix, ((0, 0), (0, _DPAD - _DIM)))
    out = _gather(
        idx_t.reshape(_HALF // _IW, _IW),
        idx_j.reshape(_HALF // _IW, _IW),
        table_padded,
    )
    return out.reshape(2, _BATCH, _MAX_LEN, _DIM)


# speedup vs baseline: 1.1053x; 1.1053x over previous
import functools

import jax
import jax.numpy as jnp
from jax import lax
from jax.experimental import pallas as pl
from jax.experimental.pallas import tpu as pltpu
from jax.experimental.pallas import tpu_sc as plsc

_BATCH = 4096
_MAX_LEN = 200
_DIM = 50
_DPAD = 64
_NW = 32
_LT = _MAX_LEN // 8
_BT = _BATCH // 128
_PATCH = 1024


def _make_gather():
    mesh = plsc.VectorSubcoreMesh(core_axis_name="c", subcore_axis_name="s")

    @functools.partial(
        pl.kernel,
        out_type=jax.ShapeDtypeStruct((2, _DIM, _LT, _BT, 8, 128),
                                      jnp.float32),
        mesh=mesh,
        scratch_types=[
            pltpu.VMEM((8, 128), jnp.int32),
            pltpu.VMEM((_PATCH, _DPAD), jnp.float32),
            pltpu.VMEM((_DIM, 8, 128), jnp.float32),
            pltpu.SemaphoreType.DMA,
        ],
        compiler_params=pltpu.CompilerParams(
            use_tc_tiling_on_sc=False, needs_layout_passes=False
        ),
    )
    def gather(idx_t_hbm, idx_j_hbm, table_hbm, out_hbm, idx_v, win_v,
               tiles_v, sem):
        wid = lax.axis_index("s") * 2 + lax.axis_index("c")
        for h, idx_hbm in enumerate((idx_t_hbm, idx_j_hbm)):

            @pl.loop(0, _LT)
            def _patch(lt, idx_hbm=idx_hbm, h=h):
                pltpu.sync_copy(
                    idx_hbm.at[pl.ds(lt * 8, 8), pl.ds(wid * 128, 128)],
                    idx_v,
                )
                copies = [
                    pltpu.async_copy(
                        table_hbm.at[idx_v.at[li]],
                        win_v.at[pl.ds(li * 128, 128)],
                        sem,
                    )
                    for li in range(8)
                ]
                for cp in copies:
                    cp.wait()

                @pl.loop(0, 8)
                def _li(li):
                    for seg in range(8):
                        row = lax.iota(jnp.int32, 16) + (li * 128 + seg * 16)
                        for d in range(_DIM):
                            vals = plsc.load_gather(
                                win_v,
                                [row, jnp.full((16,), d, jnp.int32)],
                            )
                            tiles_v[d, li, pl.ds(seg * 16, 16)] = vals

                pltpu.sync_copy(tiles_v, out_hbm.at[h, :, lt, wid])

    return gather


_gather = _make_gather()


def kernel(idx_t, idx_j, emb_matrix):
    table_padded = jnp.pad(emb_matrix, ((0, 0), (0, _DPAD - _DIM)))
    out = _gather(idx_t.T, idx_j.T, table_padded)
    return out.transpose(0, 3, 5, 2, 4, 1).reshape(
        2, _BATCH, _MAX_LEN, _DIM
    )
